# baseline (speedup 1.0000x reference)
import jax
import jax.numpy as jnp
from jax import lax
from jax.experimental import pallas as pl
from jax.experimental.pallas import tpu as pltpu
from jax.experimental.pallas import tpu_sc as plsc

NUM_ENT = 1000000
NUM_REL = 1000
DIM = 32
BATCH = 16384
LMBDA = 0.1

NC = 2
NS = 16
NW = NC * NS
B_PER_W = BATCH // NW
CHUNK = 128
N_CHUNKS = B_PER_W // CHUNK


def _sc_gather(batch_h, batch_t, batch_r,
               emb_s_a, emb_x_a, emb_y_a, emb_z_a,
               rel_s_b, rel_x_b, rel_y_b, rel_z_b):
    mesh = plsc.VectorSubcoreMesh(core_axis_name="c", subcore_axis_name="s",
                                  num_cores=NC, num_subcores=NS)
    out_t = [jax.ShapeDtypeStruct((BATCH, DIM), jnp.float32) for _ in range(12)]
    scratch = (
        [pltpu.VMEM((CHUNK,), jnp.int32) for _ in range(3)]
        + [pltpu.VMEM((CHUNK, DIM), jnp.float32) for _ in range(12)]
        + [pltpu.SemaphoreType.DMA]
    )

    def body(h_hbm, t_hbm, r_hbm,
             es_hbm, ex_hbm, ey_hbm, ez_hbm,
             rs_hbm, rx_hbm, ry_hbm, rz_hbm,
             o_sa, o_xa, o_ya, o_za, o_sc, o_xc, o_yc, o_zc,
             o_sb, o_xb, o_yb, o_zb,
             ih_v, it_v, ir_v,
             b_sa, b_xa, b_ya, b_za, b_sc, b_xc, b_yc, b_zc,
             b_sb, b_xb, b_yb, b_zb, sem):
        wid = lax.axis_index("s") * NC + lax.axis_index("c")
        base = wid * B_PER_W
        ent_tabs = (es_hbm, ex_hbm, ey_hbm, ez_hbm)
        h_bufs = (b_sa, b_xa, b_ya, b_za)
        t_bufs = (b_sc, b_xc, b_yc, b_zc)
        rel_tabs = (rs_hbm, rx_hbm, ry_hbm, rz_hbm)
        r_bufs = (b_sb, b_xb, b_yb, b_zb)
        outs = (o_sa, o_xa, o_ya, o_za, o_sc, o_xc, o_yc, o_zc,
                o_sb, o_xb, o_yb, o_zb)
        for ch in range(N_CHUNKS):
            off = base + ch * CHUNK
            pltpu.sync_copy(h_hbm.at[pl.ds(off, CHUNK)], ih_v)
            pltpu.sync_copy(t_hbm.at[pl.ds(off, CHUNK)], it_v)
            pltpu.sync_copy(r_hbm.at[pl.ds(off, CHUNK)], ir_v)
            descs = []
            for tab, buf in zip(ent_tabs, h_bufs):
                descs.append(pltpu.async_copy(tab.at[ih_v], buf, sem))
            for tab, buf in zip(ent_tabs, t_bufs):
                descs.append(pltpu.async_copy(tab.at[it_v], buf, sem))
            for tab, buf in zip(rel_tabs, r_bufs):
                descs.append(pltpu.async_copy(tab.at[ir_v], buf, sem))
            for dsc in descs:
                dsc.wait()
            for out, buf in zip(outs, h_bufs + t_bufs + r_bufs):
                pltpu.sync_copy(buf, out.at[pl.ds(off, CHUNK)])

    return pl.kernel(body, out_type=out_t, mesh=mesh, scratch_types=scratch,
                     compiler_params=pltpu.CompilerParams(
                         use_tc_tiling_on_sc=False))(
        batch_h, batch_t, batch_r,
        emb_s_a, emb_x_a, emb_y_a, emb_z_a,
        rel_s_b, rel_x_b, rel_y_b, rel_z_b)


TC_BLK = 1024
TC_GRID = BATCH // TC_BLK


def _tc_loss_body(y_ref, sa, xa, ya, za, sc_, xc, yc, zc, sb, xb, yb, zb,
                  out_ref):
    i = pl.program_id(0)

    s_a, x_a, y_a, z_a = sa[...], xa[...], ya[...], za[...]
    s_c, x_c, y_c, z_c = sc_[...], xc[...], yc[...], zc[...]
    s_b, x_b, y_b, z_b = sb[...], xb[...], yb[...], zb[...]

    rel_sq = s_b * s_b + x_b * x_b + y_b * y_b + z_b * z_b
    inv = lax.rsqrt(rel_sq)
    ns, nx, ny, nz = s_b * inv, x_b * inv, y_b * inv, z_b * inv

    A = s_a * ns - x_a * nx - y_a * ny - z_a * nz
    Bq = s_a * nx + ns * x_a + y_a * nz - ny * z_a
    Cq = s_a * ny + ns * y_a + z_a * nx - nz * x_a
    Dq = s_a * nz + ns * z_a + x_a * ny - nx * y_a
    score = -jnp.sum(A * s_c + Bq * x_c + Cq * y_c + Dq * z_c, axis=-1)

    y = y_ref[:, 0]
    sy = score * y
    softplus_sum = jnp.sum(jnp.log1p(jnp.exp(-jnp.abs(sy))) + jnp.maximum(sy, 0.0))

    ent_sq = (s_a * s_a + x_a * x_a + y_a * y_a + z_a * z_a
              + s_c * s_c + x_c * x_c + y_c * y_c + z_c * z_c)
    part = (softplus_sum / BATCH
            + (LMBDA / (BATCH * DIM)) * (jnp.sum(ent_sq) + jnp.sum(rel_sq)))

    @pl.when(i == 0)
    def _():
        out_ref[0, 0] = 0.0

    out_ref[0, 0] += part


def _tc_loss(batch_y, gathered):
    y2 = batch_y.reshape(BATCH, 1)
    row_spec = pl.BlockSpec((TC_BLK, DIM), lambda i: (i, 0))
    y_spec = pl.BlockSpec((TC_BLK, 1), lambda i: (i, 0))
    out = pl.pallas_call(
        _tc_loss_body,
        grid=(TC_GRID,),
        in_specs=[y_spec] + [row_spec] * 12,
        out_specs=pl.BlockSpec(memory_space=pltpu.SMEM),
        out_shape=jax.ShapeDtypeStruct((1, 1), jnp.float32),
    )(y2, *gathered)
    return out[0, 0]


def kernel(batch_h, batch_t, batch_r, batch_y,
           emb_s_a, emb_x_a, emb_y_a, emb_z_a,
           rel_s_b, rel_x_b, rel_y_b, rel_z_b):
    gathered = _sc_gather(batch_h, batch_t, batch_r,
                          emb_s_a, emb_x_a, emb_y_a, emb_z_a,
                          rel_s_b, rel_x_b, rel_y_b, rel_z_b)
    return _tc_loss(batch_y, gathered)


# speedup vs baseline: 1.1808x; 1.1808x over previous
import jax
import jax.numpy as jnp
from jax import lax
from jax.experimental import pallas as pl
from jax.experimental.pallas import tpu as pltpu
from jax.experimental.pallas import tpu_sc as plsc

NUM_ENT = 1000000
NUM_REL = 1000
DIM = 32
BATCH = 16384
LMBDA = 0.1

NC = 2
NS = 16
NW = NC * NS
B_PER_W = BATCH // NW
CHUNK = 256
N_CHUNKS = B_PER_W // CHUNK
FDIM = 4 * DIM


def _sc_gather(batch_h, batch_t, batch_r, ent4, rel4):
    mesh = plsc.VectorSubcoreMesh(core_axis_name="c", subcore_axis_name="s",
                                  num_cores=NC, num_subcores=NS)
    out_t = [jax.ShapeDtypeStruct((BATCH, FDIM), jnp.float32) for _ in range(3)]
    scratch = (
        [pltpu.VMEM((CHUNK,), jnp.int32) for _ in range(3)]
        + [pltpu.VMEM((CHUNK, FDIM), jnp.float32) for _ in range(3)]
        + [pltpu.SemaphoreType.DMA]
    )

    def body(h_hbm, t_hbm, r_hbm, ent_hbm, rel_hbm,
             o_h, o_t, o_r,
             ih_v, it_v, ir_v, b_h, b_t, b_r, sem):
        wid = lax.axis_index("s") * NC + lax.axis_index("c")
        base = wid * B_PER_W
        for ch in range(N_CHUNKS):
            off = base + ch * CHUNK
            pltpu.sync_copy(h_hbm.at[pl.ds(off, CHUNK)], ih_v)
            pltpu.sync_copy(t_hbm.at[pl.ds(off, CHUNK)], it_v)
            pltpu.sync_copy(r_hbm.at[pl.ds(off, CHUNK)], ir_v)
            descs = [
                pltpu.async_copy(ent_hbm.at[ih_v], b_h, sem),
                pltpu.async_copy(ent_hbm.at[it_v], b_t, sem),
                pltpu.async_copy(rel_hbm.at[ir_v], b_r, sem),
            ]
            for d in descs:
                d.wait()
            pltpu.sync_copy(b_h, o_h.at[pl.ds(off, CHUNK)])
            pltpu.sync_copy(b_t, o_t.at[pl.ds(off, CHUNK)])
            pltpu.sync_copy(b_r, o_r.at[pl.ds(off, CHUNK)])

    return pl.kernel(body, out_type=out_t, mesh=mesh, scratch_types=scratch)(
        batch_h, batch_t, batch_r, ent4, rel4)


TC_BLK = 1024
TC_GRID = BATCH // TC_BLK


def _tc_loss_body(y_ref, gh, gt, gr, out_ref):
    i = pl.program_id(0)

    h = gh[...]
    t = gt[...]
    r = gr[...]
    s_a, x_a, y_a, z_a = (h[:, 0:32], h[:, 32:64], h[:, 64:96], h[:, 96:128])
    s_c, x_c, y_c, z_c = (t[:, 0:32], t[:, 32:64], t[:, 64:96], t[:, 96:128])
    s_b, x_b, y_b, z_b = (r[:, 0:32], r[:, 32:64], r[:, 64:96], r[:, 96:128])

    rel_sq = s_b * s_b + x_b * x_b + y_b * y_b + z_b * z_b
    inv = lax.rsqrt(rel_sq)
    ns, nx, ny, nz = s_b * inv, x_b * inv, y_b * inv, z_b * inv

    A = s_a * ns - x_a * nx - y_a * ny - z_a * nz
    Bq = s_a * nx + ns * x_a + y_a * nz - ny * z_a
    Cq = s_a * ny + ns * y_a + z_a * nx - nz * x_a
    Dq = s_a * nz + ns * z_a + x_a * ny - nx * y_a
    score = -jnp.sum(A * s_c + Bq * x_c + Cq * y_c + Dq * z_c, axis=-1)

    y = y_ref[:, 0]
    sy = score * y
    softplus_sum = jnp.sum(jnp.log1p(jnp.exp(-jnp.abs(sy))) + jnp.maximum(sy, 0.0))

    ent_sq = h * h + t * t
    part = (softplus_sum / BATCH
            + (LMBDA / (BATCH * DIM)) * (jnp.sum(ent_sq) + jnp.sum(rel_sq)))

    @pl.when(i == 0)
    def _():
        out_ref[0, 0] = 0.0

    out_ref[0, 0] += part


def _tc_loss(batch_y, gh, gt, gr):
    y2 = batch_y.reshape(BATCH, 1)
    row_spec = pl.BlockSpec((TC_BLK, FDIM), lambda i: (i, 0))
    y_spec = pl.BlockSpec((TC_BLK, 1), lambda i: (i, 0))
    out = pl.pallas_call(
        _tc_loss_body,
        grid=(TC_GRID,),
        in_specs=[y_spec] + [row_spec] * 3,
        out_specs=pl.BlockSpec(memory_space=pltpu.SMEM),
        out_shape=jax.ShapeDtypeStruct((1, 1), jnp.float32),
    )(y2, gh, gt, gr)
    return out[0, 0]


def kernel(batch_h, batch_t, batch_r, batch_y,
           emb_s_a, emb_x_a, emb_y_a, emb_z_a,
           rel_s_b, rel_x_b, rel_y_b, rel_z_b):
    ent4 = jnp.concatenate([emb_s_a, emb_x_a, emb_y_a, emb_z_a], axis=1)
    rel4 = jnp.concatenate([rel_s_b, rel_x_b, rel_y_b, rel_z_b], axis=1)
    gh, gt, gr = _sc_gather(batch_h, batch_t, batch_r, ent4, rel4)
    return _tc_loss(batch_y, gh, gt, gr)


# speedup vs baseline: 1.6562x; 1.4026x over previous
import jax
import jax.numpy as jnp
from jax import lax
from jax.experimental import pallas as pl
from jax.experimental.pallas import tpu as pltpu
from jax.experimental.pallas import tpu_sc as plsc

NUM_ENT = 1000000
NUM_REL = 1000
DIM = 32
BATCH = 16384
LMBDA = 0.1

NC = 2
NS = 16
NW = NC * NS
B_PER_W = BATCH // NW
CHUNK = 256
N_CHUNKS = B_PER_W // CHUNK
FDIM = 4 * DIM


def _sc_gather(batch_h, batch_t, batch_r, ent4, rel4):
    mesh = plsc.VectorSubcoreMesh(core_axis_name="c", subcore_axis_name="s",
                                  num_cores=NC, num_subcores=NS)
    out_t = [jax.ShapeDtypeStruct((BATCH, FDIM), jnp.float32) for _ in range(3)]
    scratch = (
        [pltpu.VMEM((CHUNK,), jnp.int32) for _ in range(3)]
        + [pltpu.VMEM((CHUNK, FDIM), jnp.float32) for _ in range(3)]
        + [pltpu.SemaphoreType.DMA]
    )

    def body(h_hbm, t_hbm, r_hbm, ent_hbm, rel_hbm,
             o_h, o_t, o_r,
             ih_v, it_v, ir_v, b_h, b_t, b_r, sem):
        wid = lax.axis_index("s") * NC + lax.axis_index("c")
        base = wid * B_PER_W
        for ch in range(N_CHUNKS):
            off = base + ch * CHUNK
            pltpu.sync_copy(h_hbm.at[pl.ds(off, CHUNK)], ih_v)
            pltpu.sync_copy(t_hbm.at[pl.ds(off, CHUNK)], it_v)
            pltpu.sync_copy(r_hbm.at[pl.ds(off, CHUNK)], ir_v)
            descs = [
                pltpu.async_copy(ent_hbm.at[ih_v], b_h, sem),
                pltpu.async_copy(ent_hbm.at[it_v], b_t, sem),
                pltpu.async_copy(rel_hbm.at[ir_v], b_r, sem),
            ]
            for d in descs:
                d.wait()
            pltpu.sync_copy(b_h, o_h.at[pl.ds(off, CHUNK)])
            pltpu.sync_copy(b_t, o_t.at[pl.ds(off, CHUNK)])
            pltpu.sync_copy(b_r, o_r.at[pl.ds(off, CHUNK)])

    return pl.kernel(body, out_type=out_t, mesh=mesh, scratch_types=scratch)(
        batch_h, batch_t, batch_r, ent4, rel4)


PACK_E = 8192
PACK_GRID = -(-NUM_ENT // PACK_E)


def _tc_pack_body(s_ref, x_ref, y_ref, z_ref, out_ref):
    out_ref[:, 0:32] = s_ref[...].T
    out_ref[:, 32:64] = x_ref[...].T
    out_ref[:, 64:96] = y_ref[...].T
    out_ref[:, 96:128] = z_ref[...].T


def _tc_pack(sT, xT, yT, zT):
    in_spec = pl.BlockSpec((DIM, PACK_E), lambda i: (0, i))
    return pl.pallas_call(
        _tc_pack_body,
        grid=(PACK_GRID,),
        in_specs=[in_spec] * 4,
        out_specs=pl.BlockSpec((PACK_E, FDIM), lambda i: (i, 0)),
        out_shape=jax.ShapeDtypeStruct((NUM_ENT, FDIM), jnp.float32),
    )(sT, xT, yT, zT)


TC_BLK = 1024
TC_GRID = BATCH // TC_BLK


def _tc_loss_body(y_ref, gh, gt, gr, out_ref):
    i = pl.program_id(0)

    h = gh[...]
    t = gt[...]
    r = gr[...]
    s_a, x_a, y_a, z_a = (h[:, 0:32], h[:, 32:64], h[:, 64:96], h[:, 96:128])
    s_c, x_c, y_c, z_c = (t[:, 0:32], t[:, 32:64], t[:, 64:96], t[:, 96:128])
    s_b, x_b, y_b, z_b = (r[:, 0:32], r[:, 32:64], r[:, 64:96], r[:, 96:128])

    rel_sq = s_b * s_b + x_b * x_b + y_b * y_b + z_b * z_b
    inv = lax.rsqrt(rel_sq)
    ns, nx, ny, nz = s_b * inv, x_b * inv, y_b * inv, z_b * inv

    A = s_a * ns - x_a * nx - y_a * ny - z_a * nz
    Bq = s_a * nx + ns * x_a + y_a * nz - ny * z_a
    Cq = s_a * ny + ns * y_a + z_a * nx - nz * x_a
    Dq = s_a * nz + ns * z_a + x_a * ny - nx * y_a
    score = -jnp.sum(A * s_c + Bq * x_c + Cq * y_c + Dq * z_c, axis=-1)

    y = y_ref[:, 0]
    sy = score * y
    softplus_sum = jnp.sum(jnp.log1p(jnp.exp(-jnp.abs(sy))) + jnp.maximum(sy, 0.0))

    ent_sq = h * h + t * t
    part = (softplus_sum / BATCH
            + (LMBDA / (BATCH * DIM)) * (jnp.sum(ent_sq) + jnp.sum(rel_sq)))

    @pl.when(i == 0)
    def _():
        out_ref[0, 0] = 0.0

    out_ref[0, 0] += part


def _tc_loss(batch_y, gh, gt, gr):
    y2 = batch_y.reshape(BATCH, 1)
    row_spec = pl.BlockSpec((TC_BLK, FDIM), lambda i: (i, 0))
    y_spec = pl.BlockSpec((TC_BLK, 1), lambda i: (i, 0))
    out = pl.pallas_call(
        _tc_loss_body,
        grid=(TC_GRID,),
        in_specs=[y_spec] + [row_spec] * 3,
        out_specs=pl.BlockSpec(memory_space=pltpu.SMEM),
        out_shape=jax.ShapeDtypeStruct((1, 1), jnp.float32),
    )(y2, gh, gt, gr)
    return out[0, 0]


def kernel(batch_h, batch_t, batch_r, batch_y,
           emb_s_a, emb_x_a, emb_y_a, emb_z_a,
           rel_s_b, rel_x_b, rel_y_b, rel_z_b):
    ent4 = _tc_pack(emb_s_a.T, emb_x_a.T, emb_y_a.T, emb_z_a.T)
    rel4 = jnp.concatenate([rel_s_b, rel_x_b, rel_y_b, rel_z_b], axis=1)
    gh, gt, gr = _sc_gather(batch_h, batch_t, batch_r, ent4, rel4)
    return _tc_loss(batch_y, gh, gt, gr)


# speedup vs baseline: 3.8961x; 2.3524x over previous
import jax
import jax.numpy as jnp
from jax import lax
from jax.experimental import pallas as pl
from jax.experimental.pallas import tpu as pltpu
from jax.experimental.pallas import tpu_sc as plsc

NUM_ENT = 1000000
NUM_REL = 1000
DIM = 32
BATCH = 16384
LMBDA = 0.1

NC = 2
NS = 16
NW = NC * NS
B_PER_W = BATCH // NW
CHUNK = 256
N_CHUNKS = B_PER_W // CHUNK
FDIM = 4 * DIM


def _sc_gather(batch_h, batch_t, batch_r, ent4, rel4):
    mesh = plsc.VectorSubcoreMesh(core_axis_name="c", subcore_axis_name="s",
                                  num_cores=NC, num_subcores=NS)
    out_t = [jax.ShapeDtypeStruct((BATCH, FDIM), jnp.float32) for _ in range(3)]
    scratch = (
        [pltpu.VMEM((CHUNK,), jnp.int32) for _ in range(3)]
        + [pltpu.VMEM((CHUNK, FDIM), jnp.float32) for _ in range(3)]
        + [pltpu.SemaphoreType.DMA]
    )

    def body(h_hbm, t_hbm, r_hbm, ent_hbm, rel_hbm,
             o_h, o_t, o_r,
             ih_v, it_v, ir_v, b_h, b_t, b_r, sem):
        wid = lax.axis_index("s") * NC + lax.axis_index("c")
        base = wid * B_PER_W
        for ch in range(N_CHUNKS):
            off = base + ch * CHUNK
            pltpu.sync_copy(h_hbm.at[pl.ds(off, CHUNK)], ih_v)
            pltpu.sync_copy(t_hbm.at[pl.ds(off, CHUNK)], it_v)
            pltpu.sync_copy(r_hbm.at[pl.ds(off, CHUNK)], ir_v)
            descs = [
                pltpu.async_copy(ent_hbm.at[ih_v], b_h, sem),
                pltpu.async_copy(ent_hbm.at[it_v], b_t, sem),
                pltpu.async_copy(rel_hbm.at[ir_v], b_r, sem),
            ]
            for d in descs:
                d.wait()
            pltpu.sync_copy(b_h, o_h.at[pl.ds(off, CHUNK)])
            pltpu.sync_copy(b_t, o_t.at[pl.ds(off, CHUNK)])
            pltpu.sync_copy(b_r, o_r.at[pl.ds(off, CHUNK)])

    return pl.kernel(body, out_type=out_t, mesh=mesh, scratch_types=scratch)(
        batch_h, batch_t, batch_r, ent4, rel4)


PACK_E = 8192
PACK_GRID = -(-NUM_ENT // PACK_E)


def _tc_pack_body(s_ref, x_ref, y_ref, z_ref, out_ref):
    r = jax.lax.broadcasted_iota(jnp.int32, (FDIM, FDIM), 0)
    c = jax.lax.broadcasted_iota(jnp.int32, (FDIM, FDIM), 1)
    eye = jnp.where(r == c, 1.0, 0.0).astype(jnp.float32)
    blk = jnp.concatenate(
        [s_ref[...], x_ref[...], y_ref[...], z_ref[...]], axis=0)
    out_ref[...] = jax.lax.dot_general(
        blk, eye, (((0,), (0,)), ((), ())),
        preferred_element_type=jnp.float32)


def _tc_pack(sT, xT, yT, zT):
    in_spec = pl.BlockSpec((DIM, PACK_E), lambda i: (0, i))
    return pl.pallas_call(
        _tc_pack_body,
        grid=(PACK_GRID,),
        in_specs=[in_spec] * 4,
        out_specs=pl.BlockSpec((PACK_E, FDIM), lambda i: (i, 0)),
        out_shape=jax.ShapeDtypeStruct((NUM_ENT, FDIM), jnp.float32),
    )(sT, xT, yT, zT)


TC_BLK = 1024
TC_GRID = BATCH // TC_BLK


def _tc_loss_body(y_ref, gh, gt, gr, out_ref):
    i = pl.program_id(0)

    h = gh[...]
    t = gt[...]
    r = gr[...]
    s_a, x_a, y_a, z_a = (h[:, 0:32], h[:, 32:64], h[:, 64:96], h[:, 96:128])
    s_c, x_c, y_c, z_c = (t[:, 0:32], t[:, 32:64], t[:, 64:96], t[:, 96:128])
    s_b, x_b, y_b, z_b = (r[:, 0:32], r[:, 32:64], r[:, 64:96], r[:, 96:128])

    rel_sq = s_b * s_b + x_b * x_b + y_b * y_b + z_b * z_b
    inv = lax.rsqrt(rel_sq)
    ns, nx, ny, nz = s_b * inv, x_b * inv, y_b * inv, z_b * inv

    A = s_a * ns - x_a * nx - y_a * ny - z_a * nz
    Bq = s_a * nx + ns * x_a + y_a * nz - ny * z_a
    Cq = s_a * ny + ns * y_a + z_a * nx - nz * x_a
    Dq = s_a * nz + ns * z_a + x_a * ny - nx * y_a
    score = -jnp.sum(A * s_c + Bq * x_c + Cq * y_c + Dq * z_c, axis=-1)

    y = y_ref[:, 0]
    sy = score * y
    softplus_sum = jnp.sum(jnp.log1p(jnp.exp(-jnp.abs(sy))) + jnp.maximum(sy, 0.0))

    ent_sq = h * h + t * t
    part = (softplus_sum / BATCH
            + (LMBDA / (BATCH * DIM)) * (jnp.sum(ent_sq) + jnp.sum(rel_sq)))

    @pl.when(i == 0)
    def _():
        out_ref[0, 0] = 0.0

    out_ref[0, 0] += part


def _tc_loss(batch_y, gh, gt, gr):
    y2 = batch_y.reshape(BATCH, 1)
    row_spec = pl.BlockSpec((TC_BLK, FDIM), lambda i: (i, 0))
    y_spec = pl.BlockSpec((TC_BLK, 1), lambda i: (i, 0))
    out = pl.pallas_call(
        _tc_loss_body,
        grid=(TC_GRID,),
        in_specs=[y_spec] + [row_spec] * 3,
        out_specs=pl.BlockSpec(memory_space=pltpu.SMEM),
        out_shape=jax.ShapeDtypeStruct((1, 1), jnp.float32),
    )(y2, gh, gt, gr)
    return out[0, 0]


def kernel(batch_h, batch_t, batch_r, batch_y,
           emb_s_a, emb_x_a, emb_y_a, emb_z_a,
           rel_s_b, rel_x_b, rel_y_b, rel_z_b):
    ent4 = _tc_pack(emb_s_a.T, emb_x_a.T, emb_y_a.T, emb_z_a.T)
    rel4 = jnp.concatenate([rel_s_b, rel_x_b, rel_y_b, rel_z_b], axis=1)
    gh, gt, gr = _sc_gather(batch_h, batch_t, batch_r, ent4, rel4)
    return _tc_loss(batch_y, gh, gt, gr)


# speedup vs baseline: 3.9718x; 1.0194x over previous
import jax
import jax.numpy as jnp
from jax import lax
from jax.experimental import pallas as pl
from jax.experimental.pallas import tpu as pltpu
from jax.experimental.pallas import tpu_sc as plsc

NUM_ENT = 1000000
NUM_REL = 1000
DIM = 32
BATCH = 16384
LMBDA = 0.1

NC = 2
NS = 16
NW = NC * NS
B_PER_W = BATCH // NW
CHUNK = 256
N_CHUNKS = B_PER_W // CHUNK
FDIM = 4 * DIM


def _sc_gather(batch_h, batch_t, batch_r, ent4, rel4):
    mesh = plsc.VectorSubcoreMesh(core_axis_name="c", subcore_axis_name="s",
                                  num_cores=NC, num_subcores=NS)
    out_t = [jax.ShapeDtypeStruct((BATCH, FDIM), jnp.float32) for _ in range(3)]
    scratch = (
        [pltpu.VMEM((CHUNK,), jnp.int32) for _ in range(3)]
        + [pltpu.VMEM((CHUNK, FDIM), jnp.float32) for _ in range(3)]
        + [pltpu.SemaphoreType.DMA]
    )

    def body(h_hbm, t_hbm, r_hbm, ent_hbm, rel_hbm,
             o_h, o_t, o_r,
             ih_v, it_v, ir_v, b_h, b_t, b_r, sem):
        wid = lax.axis_index("s") * NC + lax.axis_index("c")
        base = wid * B_PER_W
        for ch in range(N_CHUNKS):
            off = base + ch * CHUNK
            pltpu.sync_copy(h_hbm.at[pl.ds(off, CHUNK)], ih_v)
            pltpu.sync_copy(t_hbm.at[pl.ds(off, CHUNK)], it_v)
            pltpu.sync_copy(r_hbm.at[pl.ds(off, CHUNK)], ir_v)
            descs = [
                pltpu.async_copy(ent_hbm.at[ih_v], b_h, sem),
                pltpu.async_copy(ent_hbm.at[it_v], b_t, sem),
                pltpu.async_copy(rel_hbm.at[ir_v], b_r, sem),
            ]
            for d in descs:
                d.wait()
            pltpu.sync_copy(b_h, o_h.at[pl.ds(off, CHUNK)])
            pltpu.sync_copy(b_t, o_t.at[pl.ds(off, CHUNK)])
            pltpu.sync_copy(b_r, o_r.at[pl.ds(off, CHUNK)])

    return pl.kernel(body, out_type=out_t, mesh=mesh, scratch_types=scratch)(
        batch_h, batch_t, batch_r, ent4, rel4)


PACK_E = 16384
PACK_GRID = -(-NUM_ENT // PACK_E)


def _tc_pack_body(s_ref, x_ref, y_ref, z_ref, out_ref):
    r = jax.lax.broadcasted_iota(jnp.int32, (FDIM, FDIM), 0)
    c = jax.lax.broadcasted_iota(jnp.int32, (FDIM, FDIM), 1)
    eye = jnp.where(r == c, 1.0, 0.0).astype(jnp.float32)
    blk = jnp.concatenate(
        [s_ref[...], x_ref[...], y_ref[...], z_ref[...]], axis=0)
    out_ref[...] = jax.lax.dot_general(
        blk, eye, (((0,), (0,)), ((), ())),
        preferred_element_type=jnp.float32)


def _tc_pack(sT, xT, yT, zT):
    in_spec = pl.BlockSpec((DIM, PACK_E), lambda i: (0, i))
    return pl.pallas_call(
        _tc_pack_body,
        grid=(PACK_GRID,),
        in_specs=[in_spec] * 4,
        out_specs=pl.BlockSpec((PACK_E, FDIM), lambda i: (i, 0)),
        out_shape=jax.ShapeDtypeStruct((NUM_ENT, FDIM), jnp.float32),
    )(sT, xT, yT, zT)


TC_BLK = 1024
TC_GRID = BATCH // TC_BLK


def _tc_loss_body(y_ref, gh, gt, gr, out_ref):
    i = pl.program_id(0)

    h = gh[...]
    t = gt[...]
    r = gr[...]
    s_a, x_a, y_a, z_a = (h[:, 0:32], h[:, 32:64], h[:, 64:96], h[:, 96:128])
    s_c, x_c, y_c, z_c = (t[:, 0:32], t[:, 32:64], t[:, 64:96], t[:, 96:128])
    s_b, x_b, y_b, z_b = (r[:, 0:32], r[:, 32:64], r[:, 64:96], r[:, 96:128])

    rel_sq = s_b * s_b + x_b * x_b + y_b * y_b + z_b * z_b
    inv = lax.rsqrt(rel_sq)
    ns, nx, ny, nz = s_b * inv, x_b * inv, y_b * inv, z_b * inv

    A = s_a * ns - x_a * nx - y_a * ny - z_a * nz
    Bq = s_a * nx + ns * x_a + y_a * nz - ny * z_a
    Cq = s_a * ny + ns * y_a + z_a * nx - nz * x_a
    Dq = s_a * nz + ns * z_a + x_a * ny - nx * y_a
    score = -jnp.sum(A * s_c + Bq * x_c + Cq * y_c + Dq * z_c, axis=-1)

    y = y_ref[:, 0]
    sy = score * y
    softplus_sum = jnp.sum(jnp.log1p(jnp.exp(-jnp.abs(sy))) + jnp.maximum(sy, 0.0))

    ent_sq = h * h + t * t
    part = (softplus_sum / BATCH
            + (LMBDA / (BATCH * DIM)) * (jnp.sum(ent_sq) + jnp.sum(rel_sq)))

    @pl.when(i == 0)
    def _():
        out_ref[0, 0] = 0.0

    out_ref[0, 0] += part


def _tc_loss(batch_y, gh, gt, gr):
    y2 = batch_y.reshape(BATCH, 1)
    row_spec = pl.BlockSpec((TC_BLK, FDIM), lambda i: (i, 0))
    y_spec = pl.BlockSpec((TC_BLK, 1), lambda i: (i, 0))
    out = pl.pallas_call(
        _tc_loss_body,
        grid=(TC_GRID,),
        in_specs=[y_spec] + [row_spec] * 3,
        out_specs=pl.BlockSpec(memory_space=pltpu.SMEM),
        out_shape=jax.ShapeDtypeStruct((1, 1), jnp.float32),
    )(y2, gh, gt, gr)
    return out[0, 0]


def kernel(batch_h, batch_t, batch_r, batch_y,
           emb_s_a, emb_x_a, emb_y_a, emb_z_a,
           rel_s_b, rel_x_b, rel_y_b, rel_z_b):
    ent4 = _tc_pack(emb_s_a.T, emb_x_a.T, emb_y_a.T, emb_z_a.T)
    rel4 = jnp.concatenate([rel_s_b, rel_x_b, rel_y_b, rel_z_b], axis=1)
    gh, gt, gr = _sc_gather(batch_h, batch_t, batch_r, ent4, rel4)
    return _tc_loss(batch_y, gh, gt, gr)
